# Initial kernel scaffold; baseline (speedup 1.0000x reference)
import jax
import jax.numpy as jnp
from jax.experimental import pallas as pl


def kernel(x, edge_index, W1, b1, W2, b2, W_out, b_out):
    raise NotImplementedError("write your pallas kernel here")


# speedup vs baseline: 8.5812x; 8.5812x over previous
import functools

import jax
import jax.numpy as jnp
from jax import lax
from jax.experimental import pallas as pl
from jax.experimental.pallas import tpu as pltpu
from jax.experimental.pallas import tpu_sc as plsc

_N = 10000
_E = 320000
_F = 128
_CH = 128
_TCH = _E // _CH
_NC = 2
_NS = 16
_NW = _NC * _NS
_RPT = (_N // _NS) // 8 * 8
_RTAIL = _N - _NS * _RPT
_BM = 1000

_mesh = plsc.VectorSubcoreMesh(core_axis_name="c", subcore_axis_name="s")


@functools.partial(
    pl.kernel,
    mesh=_mesh,
    out_type=jax.ShapeDtypeStruct((_NC, _N, _F), jnp.float32),
    scratch_types=[
        pltpu.VMEM((_CH,), jnp.int32),
        pltpu.VMEM((_CH, _F), jnp.float32),
        pltpu.VMEM_SHARED((_N, _F), jnp.float32),
    ],
)
def _sc_degree(dst_hbm, ones_hbm, zeros_hbm, out_hbm, dst_v, ones_v, acc):
    cid = lax.axis_index("c")
    sid = lax.axis_index("s")
    wid = sid * _NC + cid
    pltpu.sync_copy(zeros_hbm.at[pl.ds(sid * _RPT, _RPT)],
                    acc.at[pl.ds(sid * _RPT, _RPT)])

    @pl.when(sid == 0)
    def _():
        pltpu.sync_copy(zeros_hbm.at[pl.ds(_NS * _RPT, _RTAIL)],
                        acc.at[pl.ds(_NS * _RPT, _RTAIL)])

    pltpu.sync_copy(ones_hbm, ones_v)
    plsc.subcore_barrier()
    nch = (_TCH - wid + _NW - 1) // _NW

    def body(j, carry):
        off = (wid + j * _NW) * _CH
        pltpu.sync_copy(dst_hbm.at[pl.ds(off, _CH)], dst_v)
        pltpu.sync_copy(ones_v, acc.at[dst_v], add=True)
        return carry

    lax.fori_loop(0, nch, body, 0)
    plsc.subcore_barrier()
    pltpu.sync_copy(acc.at[pl.ds(sid * _RPT, _RPT)],
                    out_hbm.at[cid, pl.ds(sid * _RPT, _RPT)])

    @pl.when(sid == 0)
    def _():
        pltpu.sync_copy(acc.at[pl.ds(_NS * _RPT, _RTAIL)],
                        out_hbm.at[cid, pl.ds(_NS * _RPT, _RTAIL)])


@functools.partial(
    pl.kernel,
    mesh=_mesh,
    out_type=jax.ShapeDtypeStruct((_NC, _N, _F), jnp.float32),
    scratch_types=[
        pltpu.VMEM((_CH,), jnp.int32),
        pltpu.VMEM((_CH,), jnp.int32),
        pltpu.VMEM((_CH, _F), jnp.float32),
        pltpu.VMEM_SHARED((_N, _F), jnp.float32),
        pltpu.SemaphoreType.DMA,
    ],
)
def _sc_aggregate(p_hbm, src_hbm, dst_hbm, zeros_hbm, out_hbm,
                  src_v, dst_v, rows_v, acc, sem):
    cid = lax.axis_index("c")
    sid = lax.axis_index("s")
    wid = sid * _NC + cid
    pltpu.sync_copy(zeros_hbm.at[pl.ds(sid * _RPT, _RPT)],
                    acc.at[pl.ds(sid * _RPT, _RPT)])

    @pl.when(sid == 0)
    def _():
        pltpu.sync_copy(zeros_hbm.at[pl.ds(_NS * _RPT, _RTAIL)],
                        acc.at[pl.ds(_NS * _RPT, _RTAIL)])

    plsc.subcore_barrier()
    nch = (_TCH - wid + _NW - 1) // _NW

    def body(j, carry):
        off = (wid + j * _NW) * _CH
        pltpu.sync_copy(src_hbm.at[pl.ds(off, _CH)], src_v)
        pltpu.sync_copy(dst_hbm.at[pl.ds(off, _CH)], dst_v)
        pltpu.async_copy(p_hbm.at[src_v], rows_v, sem).wait()
        pltpu.sync_copy(rows_v, acc.at[dst_v], add=True)
        return carry

    lax.fori_loop(0, nch, body, 0)
    plsc.subcore_barrier()
    pltpu.sync_copy(acc.at[pl.ds(sid * _RPT, _RPT)],
                    out_hbm.at[cid, pl.ds(sid * _RPT, _RPT)])

    @pl.when(sid == 0)
    def _():
        pltpu.sync_copy(acc.at[pl.ds(_NS * _RPT, _RTAIL)],
                        out_hbm.at[cid, pl.ds(_NS * _RPT, _RTAIL)])


def _norm_from(d0, d1):
    deg = d0[:, 0:1] + d1[:, 0:1]
    return jnp.where(deg > 0.0, lax.rsqrt(jnp.maximum(deg, 1.0)), 0.0)


def _tc_mm1_body(x_ref, w_ref, d0_ref, d1_ref, o_ref):
    norm = _norm_from(d0_ref[...], d1_ref[...])
    o_ref[...] = jnp.dot(x_ref[...], w_ref[...],
                         preferred_element_type=jnp.float32) * norm


def _tc_mid_body(a0_ref, a1_ref, d0_ref, d1_ref, b_ref, w_ref, o_ref):
    norm = _norm_from(d0_ref[...], d1_ref[...])
    h = jnp.maximum((a0_ref[...] + a1_ref[...]) * norm + b_ref[...], 0.0)
    o_ref[...] = jnp.dot(h, w_ref[...],
                         preferred_element_type=jnp.float32) * norm


def _tc_out_body(a0_ref, a1_ref, d0_ref, d1_ref, b_ref, w_ref, bo_ref, o_ref):
    norm = _norm_from(d0_ref[...], d1_ref[...])
    h = jnp.maximum((a0_ref[...] + a1_ref[...]) * norm + b_ref[...], 0.0)
    o_ref[...] = jnp.dot(h, w_ref[...],
                         preferred_element_type=jnp.float32) + bo_ref[...]


def _row_spec(w):
    return pl.BlockSpec((_BM, w), lambda i: (i, 0))


def _full_spec(shape):
    return pl.BlockSpec(shape, lambda i: (0, 0))


def kernel(x, edge_index, W1, b1, W2, b2, W_out, b_out):
    src = edge_index[0]
    dst = edge_index[1]
    nclass = W_out.shape[1]
    grid = (_N // _BM,)

    ones128 = jnp.ones((_CH, _F), jnp.float32)
    zerosf = jnp.zeros((_N, _F), jnp.float32)

    degp = _sc_degree(dst, ones128, zerosf)
    d0, d1 = degp[0], degp[1]

    p1 = pl.pallas_call(
        _tc_mm1_body,
        grid=grid,
        in_specs=[_row_spec(_F), _full_spec((_F, _F)),
                  _row_spec(_F), _row_spec(_F)],
        out_specs=_row_spec(_F),
        out_shape=jax.ShapeDtypeStruct((_N, _F), jnp.float32),
    )(x, W1, d0, d1)

    a1 = _sc_aggregate(p1, src, dst, zerosf)

    p2 = pl.pallas_call(
        _tc_mid_body,
        grid=grid,
        in_specs=[_row_spec(_F), _row_spec(_F), _row_spec(_F), _row_spec(_F),
                  _full_spec((1, _F)), _full_spec((_F, _F))],
        out_specs=_row_spec(_F),
        out_shape=jax.ShapeDtypeStruct((_N, _F), jnp.float32),
    )(a1[0], a1[1], d0, d1, b1.reshape(1, _F), W2)

    a2 = _sc_aggregate(p2, src, dst, zerosf)

    out = pl.pallas_call(
        _tc_out_body,
        grid=grid,
        in_specs=[_row_spec(_F), _row_spec(_F), _row_spec(_F), _row_spec(_F),
                  _full_spec((1, _F)), _full_spec((_F, nclass)),
                  _full_spec((1, nclass))],
        out_specs=_row_spec(nclass),
        out_shape=jax.ShapeDtypeStruct((_N, nclass), jnp.float32),
    )(a2[0], a2[1], d0, d1, b2.reshape(1, _F), W_out, b_out.reshape(1, nclass))

    return out


# speedup vs baseline: 12.4477x; 1.4506x over previous
import functools

import jax
import jax.numpy as jnp
from jax import lax
from jax.experimental import pallas as pl
from jax.experimental.pallas import tpu as pltpu
from jax.experimental.pallas import tpu_sc as plsc

_N = 10000
_E = 320000
_F = 128
_CH = 128
_TCH = _E // _CH
_NC = 2
_NS = 16
_NW = _NC * _NS
_RPT = (_N // _NS) // 8 * 8
_RTAIL = _N - _NS * _RPT
_BM = 1000

_mesh = plsc.VectorSubcoreMesh(core_axis_name="c", subcore_axis_name="s")


@functools.partial(
    pl.kernel,
    mesh=_mesh,
    out_type=jax.ShapeDtypeStruct((_NC, _N, _F), jnp.float32),
    scratch_types=[
        pltpu.VMEM((_CH,), jnp.int32),
        pltpu.VMEM((_CH, _F), jnp.float32),
        pltpu.VMEM_SHARED((_N, _F), jnp.float32),
    ],
)
def _sc_degree(dst_hbm, ones_hbm, zeros_hbm, out_hbm, dst_v, ones_v, acc):
    cid = lax.axis_index("c")
    sid = lax.axis_index("s")
    wid = sid * _NC + cid
    pltpu.sync_copy(zeros_hbm.at[pl.ds(sid * _RPT, _RPT)],
                    acc.at[pl.ds(sid * _RPT, _RPT)])

    @pl.when(sid == 0)
    def _():
        pltpu.sync_copy(zeros_hbm.at[pl.ds(_NS * _RPT, _RTAIL)],
                        acc.at[pl.ds(_NS * _RPT, _RTAIL)])

    pltpu.sync_copy(ones_hbm, ones_v)
    plsc.subcore_barrier()
    nch = (_TCH - wid + _NW - 1) // _NW

    def body(j, carry):
        off = (wid + j * _NW) * _CH
        pltpu.sync_copy(dst_hbm.at[pl.ds(off, _CH)], dst_v)
        pltpu.sync_copy(ones_v, acc.at[dst_v], add=True)
        return carry

    lax.fori_loop(0, nch, body, 0)
    plsc.subcore_barrier()
    pltpu.sync_copy(acc.at[pl.ds(sid * _RPT, _RPT)],
                    out_hbm.at[cid, pl.ds(sid * _RPT, _RPT)])

    @pl.when(sid == 0)
    def _():
        pltpu.sync_copy(acc.at[pl.ds(_NS * _RPT, _RTAIL)],
                        out_hbm.at[cid, pl.ds(_NS * _RPT, _RTAIL)])


@functools.partial(
    pl.kernel,
    mesh=_mesh,
    out_type=jax.ShapeDtypeStruct((_NC, _N, _F), jnp.float32),
    scratch_types=[
        pltpu.VMEM((_CH,), jnp.int32),
        pltpu.VMEM((_CH,), jnp.int32),
        pltpu.VMEM((_CH,), jnp.int32),
        pltpu.VMEM((_CH,), jnp.int32),
        pltpu.VMEM((_CH, _F), jnp.float32),
        pltpu.VMEM((_CH, _F), jnp.float32),
        pltpu.VMEM_SHARED((_N, _F), jnp.float32),
        pltpu.SemaphoreType.DMA,
        pltpu.SemaphoreType.DMA,
        pltpu.SemaphoreType.DMA,
        pltpu.SemaphoreType.DMA,
    ],
)
def _sc_aggregate(p_hbm, src_hbm, dst_hbm, zeros_hbm, out_hbm,
                  src0, dst0, src1, dst1, rows0, rows1, acc,
                  isem0, isem1, gsem0, gsem1):
    cid = lax.axis_index("c")
    sid = lax.axis_index("s")
    wid = sid * _NC + cid
    nch = (_TCH - wid + _NW - 1) // _NW

    def ioff(j):
        return (wid + j * _NW) * _CH

    def istart_src(j, buf, sem):
        pltpu.async_copy(src_hbm.at[pl.ds(ioff(j), _CH)], buf, sem)

    def istart_dst(j, buf, sem):
        pltpu.async_copy(dst_hbm.at[pl.ds(ioff(j), _CH)], buf, sem)

    def iwait(sbuf, dbuf, sem):
        pltpu.make_async_copy(src_hbm.at[pl.ds(0, _CH)], sbuf, sem).wait()
        pltpu.make_async_copy(dst_hbm.at[pl.ds(0, _CH)], dbuf, sem).wait()

    def gstart(sbuf, rows, sem):
        pltpu.async_copy(p_hbm.at[sbuf], rows, sem)

    def gwait(sbuf, rows, sem):
        pltpu.make_async_copy(p_hbm.at[sbuf], rows, sem).wait()

    def scat(dbuf, rows):
        pltpu.sync_copy(rows, acc.at[dbuf], add=True)

    istart_src(0, src0, isem0)
    istart_dst(0, dst0, isem0)
    istart_src(1, src1, isem1)
    istart_dst(1, dst1, isem1)

    pltpu.sync_copy(zeros_hbm.at[pl.ds(sid * _RPT, _RPT)],
                    acc.at[pl.ds(sid * _RPT, _RPT)])

    @pl.when(sid == 0)
    def _():
        pltpu.sync_copy(zeros_hbm.at[pl.ds(_NS * _RPT, _RTAIL)],
                        acc.at[pl.ds(_NS * _RPT, _RTAIL)])

    plsc.subcore_barrier()

    iwait(src0, dst0, isem0)
    gstart(src0, rows0, gsem0)
    npairs = nch // 2

    def body(p, carry):
        j0 = 2 * p
        j1 = j0 + 1
        more0 = j0 + 2 < nch
        more1 = j1 + 2 < nch

        iwait(src1, dst1, isem1)
        gstart(src1, rows1, gsem1)

        gwait(src0, rows0, gsem0)

        @pl.when(more0)
        def _():
            istart_src(j0 + 2, src0, isem0)

        scat(dst0, rows0)

        @pl.when(more0)
        def _():
            istart_dst(j0 + 2, dst0, isem0)

        gwait(src1, rows1, gsem1)

        @pl.when(more1)
        def _():
            istart_src(j1 + 2, src1, isem1)

        scat(dst1, rows1)

        @pl.when(more1)
        def _():
            istart_dst(j1 + 2, dst1, isem1)

        @pl.when(more0)
        def _():
            iwait(src0, dst0, isem0)
            gstart(src0, rows0, gsem0)

        return carry

    lax.fori_loop(0, npairs, body, 0)

    @pl.when(nch % 2 == 1)
    def _():
        gwait(src0, rows0, gsem0)
        scat(dst0, rows0)

    plsc.subcore_barrier()
    pltpu.sync_copy(acc.at[pl.ds(sid * _RPT, _RPT)],
                    out_hbm.at[cid, pl.ds(sid * _RPT, _RPT)])

    @pl.when(sid == 0)
    def _():
        pltpu.sync_copy(acc.at[pl.ds(_NS * _RPT, _RTAIL)],
                        out_hbm.at[cid, pl.ds(_NS * _RPT, _RTAIL)])


def _norm_from(d0, d1):
    deg = d0[:, 0:1] + d1[:, 0:1]
    return jnp.where(deg > 0.0, lax.rsqrt(jnp.maximum(deg, 1.0)), 0.0)


def _tc_mm1_body(x_ref, w_ref, d0_ref, d1_ref, o_ref):
    norm = _norm_from(d0_ref[...], d1_ref[...])
    o_ref[...] = jnp.dot(x_ref[...], w_ref[...],
                         preferred_element_type=jnp.float32) * norm


def _tc_mid_body(a0_ref, a1_ref, d0_ref, d1_ref, b_ref, w_ref, o_ref):
    norm = _norm_from(d0_ref[...], d1_ref[...])
    h = jnp.maximum((a0_ref[...] + a1_ref[...]) * norm + b_ref[...], 0.0)
    o_ref[...] = jnp.dot(h, w_ref[...],
                         preferred_element_type=jnp.float32) * norm


def _tc_out_body(a0_ref, a1_ref, d0_ref, d1_ref, b_ref, w_ref, bo_ref, o_ref):
    norm = _norm_from(d0_ref[...], d1_ref[...])
    h = jnp.maximum((a0_ref[...] + a1_ref[...]) * norm + b_ref[...], 0.0)
    o_ref[...] = jnp.dot(h, w_ref[...],
                         preferred_element_type=jnp.float32) + bo_ref[...]


def _row_spec(w):
    return pl.BlockSpec((_BM, w), lambda i: (i, 0))


def _full_spec(shape):
    return pl.BlockSpec(shape, lambda i: (0, 0))


def kernel(x, edge_index, W1, b1, W2, b2, W_out, b_out):
    src = edge_index[0]
    dst = edge_index[1]
    nclass = W_out.shape[1]
    grid = (_N // _BM,)

    ones128 = jnp.ones((_CH, _F), jnp.float32)
    zerosf = jnp.zeros((_N, _F), jnp.float32)

    degp = _sc_degree(dst, ones128, zerosf)
    d0, d1 = degp[0], degp[1]

    p1 = pl.pallas_call(
        _tc_mm1_body,
        grid=grid,
        in_specs=[_row_spec(_F), _full_spec((_F, _F)),
                  _row_spec(_F), _row_spec(_F)],
        out_specs=_row_spec(_F),
        out_shape=jax.ShapeDtypeStruct((_N, _F), jnp.float32),
    )(x, W1, d0, d1)

    a1 = _sc_aggregate(p1, src, dst, zerosf)

    p2 = pl.pallas_call(
        _tc_mid_body,
        grid=grid,
        in_specs=[_row_spec(_F), _row_spec(_F), _row_spec(_F), _row_spec(_F),
                  _full_spec((1, _F)), _full_spec((_F, _F))],
        out_specs=_row_spec(_F),
        out_shape=jax.ShapeDtypeStruct((_N, _F), jnp.float32),
    )(a1[0], a1[1], d0, d1, b1.reshape(1, _F), W2)

    a2 = _sc_aggregate(p2, src, dst, zerosf)

    out = pl.pallas_call(
        _tc_out_body,
        grid=grid,
        in_specs=[_row_spec(_F), _row_spec(_F), _row_spec(_F), _row_spec(_F),
                  _full_spec((1, _F)), _full_spec((_F, nclass)),
                  _full_spec((1, nclass))],
        out_specs=_row_spec(nclass),
        out_shape=jax.ShapeDtypeStruct((_N, nclass), jnp.float32),
    )(a2[0], a2[1], d0, d1, b2.reshape(1, _F), W_out, b_out.reshape(1, nclass))

    return out
